# Initial kernel scaffold; baseline (speedup 1.0000x reference)
import jax
import jax.numpy as jnp
from jax.experimental import pallas as pl


def kernel(f_plane, ix, iy, w, dl):
    raise NotImplementedError("write your pallas kernel here")


# speedup vs baseline: 234.4076x; 234.4076x over previous
import functools

import jax
import jax.numpy as jnp
from jax import lax
from jax.experimental import pallas as pl
from jax.experimental.pallas import tpu as pltpu
from jax.experimental.pallas import tpu_sc as plsc

NX, NY = 2048, 2048
N = NX * NY
K = 4
NC, NS = 2, 16
NW = NC * NS
OW = N // NW
CH = 8192
LANES = 16


@functools.partial(
    pl.kernel,
    out_type=jax.ShapeDtypeStruct((N,), jnp.float32),
    mesh=plsc.VectorSubcoreMesh(core_axis_name="c", subcore_axis_name="s"),
    scratch_types=[
        pltpu.VMEM((CH,), jnp.int32),
        pltpu.VMEM((CH,), jnp.float32),
        pltpu.VMEM((CH,), jnp.float32),
        pltpu.VMEM((CH,), jnp.float32),
        pltpu.SemaphoreType.DMA,
    ],
)
def _bilinear_sc(f_hbm, lin_hbm, w_hbm, out_hbm, idx_v, vals_v, w_v, acc_v, sem):
    wid = lax.axis_index("s") * NC + lax.axis_index("c")
    obase = wid * OW

    def chunk(c, _):
        off = obase + c * CH
        for k in range(K):
            pltpu.sync_copy(lin_hbm.at[pl.ds(k * N + off, CH)], idx_v)
            pltpu.async_copy(f_hbm.at[idx_v], vals_v, sem).wait()
            pltpu.sync_copy(w_hbm.at[pl.ds(k * N + off, CH)], w_v)

            def ew(i, _, k=k):
                s = pl.ds(i * LANES, LANES)
                p = vals_v[s] * w_v[s]
                if k == 0:
                    acc_v[s] = p
                else:
                    acc_v[s] = acc_v[s] + p
                return 0

            lax.fori_loop(0, CH // LANES, ew, 0, unroll=8)
        pltpu.sync_copy(acc_v, out_hbm.at[pl.ds(off, CH)])
        return 0

    lax.fori_loop(0, OW // CH, chunk, 0)


def kernel(f_plane, ix, iy, w, dl):
    nx, ny = f_plane.shape
    lin = ix.astype(jnp.int32) * ny + iy.astype(jnp.int32)
    lin_t = jnp.transpose(lin, (2, 0, 1)).reshape(-1)
    w_t = jnp.transpose(w, (2, 0, 1)).reshape(-1)
    out = _bilinear_sc(f_plane.reshape(-1), lin_t, w_t)
    return out.reshape(nx, ny)
